# baseline (speedup 1.0000x reference)
import functools

import jax
import jax.numpy as jnp
from jax import lax
from jax.experimental import pallas as pl
from jax.experimental.pallas import tpu as pltpu
from jax.experimental.pallas import tpu_sc as plsc

BATCH = 4096
DIM = 64
NUM_TABLES = 6

_info = plsc.get_sparse_core_info()
_NC, _NS = _info.num_cores, _info.num_subcores
_NW = _NC * _NS
_BPW = BATCH // _NW


def _make():
  mesh = plsc.VectorSubcoreMesh(core_axis_name="c", subcore_axis_name="s")

  @functools.partial(
      pl.kernel,
      mesh=mesh,
      out_type=jax.ShapeDtypeStruct((BATCH, NUM_TABLES * DIM), jnp.float32),
      compiler_params=pltpu.CompilerParams(
          use_tc_tiling_on_sc=True,
          disable_bounds_checks=True,
          disable_semaphore_checks=True,
      ),
      scratch_types=[
          pltpu.VMEM((_BPW,), jnp.int32),
          pltpu.VMEM((_BPW,), jnp.int32),
          pltpu.VMEM((2, _BPW, 2 * DIM), jnp.float32),
          pltpu.VMEM_SHARED((_NS * _BPW, 2 * DIM), jnp.float32),
          pltpu.SemaphoreType.DMA,
          pltpu.SemaphoreType.DMA,
          pltpu.SemaphoreType.DMA,
          pltpu.SemaphoreType.DMA,
      ],
  )
  def lookup(u_hbm, i_hbm, t0, t1, t2, t3, t4, t5, out_hbm,
             uid_v, iid_v, gbuf, spbuf, s0, s1, s2, wsem):
    cid = lax.axis_index("c")
    sid = lax.axis_index("s")
    wid = sid * _NC + cid
    base = wid * _BPW
    srow = sid * _BPW
    pltpu.sync_copy(u_hbm.at[pl.ds(base, _BPW)], uid_v)
    pltpu.sync_copy(i_hbm.at[pl.ds(base, _BPW)], iid_v)
    tables = (t0, t1, t2, t3, t4, t5)
    sems = (s0, s1, s2)

    def fire(p):
      def body(c, carry):
        uv = uid_v[pl.ds(c * 16, 16)]
        iv = iid_v[pl.ds(c * 16, 16)]
        for j in range(16):
          r = c * 16 + j
          for k in (2 * p, 2 * p + 1):
            idx = uv[j] if k % 2 == 0 else iv[j]
            if p == 2:
              dst = spbuf.at[srow + r, pl.ds((k % 2) * DIM, DIM)]
            else:
              dst = gbuf.at[p, r, pl.ds((k % 2) * DIM, DIM)]
            pltpu.async_copy(tables[k].at[idx], dst, sems[p])
        return carry

      lax.fori_loop(0, _BPW // 16, body, 0)

    fire(2)
    fire(0)
    fire(1)

    for p in range(2):
      pltpu.make_async_copy(
          out_hbm.at[pl.ds(base, _BPW), pl.ds(p * 2 * DIM, 2 * DIM)],
          gbuf.at[p], sems[p]).wait()
      pltpu.async_copy(
          gbuf.at[p],
          out_hbm.at[pl.ds(base, _BPW), pl.ds(p * 2 * DIM, 2 * DIM)], wsem)
    pltpu.make_async_copy(
        out_hbm.at[pl.ds(base, _BPW), pl.ds(4 * DIM, 2 * DIM)],
        spbuf.at[pl.ds(srow, _BPW)], sems[2]).wait()
    pltpu.async_copy(
        spbuf.at[pl.ds(srow, _BPW)],
        out_hbm.at[pl.ds(base, _BPW), pl.ds(4 * DIM, 2 * DIM)], wsem)
    for p in range(2):
      pltpu.make_async_copy(
          gbuf.at[p],
          out_hbm.at[pl.ds(base, _BPW), pl.ds(p * 2 * DIM, 2 * DIM)],
          wsem).wait()
    pltpu.make_async_copy(
        spbuf.at[pl.ds(srow, _BPW)],
        out_hbm.at[pl.ds(base, _BPW), pl.ds(4 * DIM, 2 * DIM)], wsem).wait()

  return lookup


_lookup = _make()


def kernel(uid, iid, user_table, item_table, src_user_0, src_item_0,
           src_user_1, src_item_1):
  return _lookup(uid.astype(jnp.int32), iid.astype(jnp.int32),
                 user_table, item_table, src_user_0, src_item_0,
                 src_user_1, src_item_1)


# speedup vs baseline: 1.1148x; 1.1148x over previous
import functools

import jax
import jax.numpy as jnp
from jax import lax
from jax.experimental import pallas as pl
from jax.experimental.pallas import tpu as pltpu
from jax.experimental.pallas import tpu_sc as plsc

BATCH = 4096
DIM = 64
NUM_TABLES = 6

_info = plsc.get_sparse_core_info()
_NC, _NS = _info.num_cores, _info.num_subcores
_NW = _NC * _NS
_BPW = BATCH // _NW


def _make():
  mesh = plsc.VectorSubcoreMesh(core_axis_name="c", subcore_axis_name="s")

  @functools.partial(
      pl.kernel,
      mesh=mesh,
      out_type=jax.ShapeDtypeStruct((BATCH, NUM_TABLES * DIM), jnp.float32),
      compiler_params=pltpu.CompilerParams(
          use_tc_tiling_on_sc=True,
          disable_bounds_checks=True,
          disable_semaphore_checks=True,
      ),
      scratch_types=[
          pltpu.VMEM((_BPW,), jnp.int32),
          pltpu.VMEM((_BPW,), jnp.int32),
          pltpu.VMEM((NUM_TABLES // 2, _BPW, 2 * DIM), jnp.float32),
          pltpu.SemaphoreType.DMA,
          pltpu.SemaphoreType.DMA,
          pltpu.SemaphoreType.DMA,
          pltpu.SemaphoreType.DMA,
      ],
  )
  def lookup(u_hbm, i_hbm, t0, t1, t2, t3, t4, t5, out_hbm,
             uid_v, iid_v, gbuf, s0, s1, s2, wsem):
    wid = lax.axis_index("s") * _NC + lax.axis_index("c")
    base = wid * _BPW
    pltpu.sync_copy(u_hbm.at[pl.ds(base, _BPW)], uid_v)
    pltpu.sync_copy(i_hbm.at[pl.ds(base, _BPW)], iid_v)
    tables = (t0, t1, t2, t3, t4, t5)
    sems = (s0, s1, s2)

    def fire(p):
      def body(c, carry):
        uv = uid_v[pl.ds(c * 16, 16)]
        iv = iid_v[pl.ds(c * 16, 16)]
        for j in range(16):
          r = c * 16 + j
          for k in (2 * p, 2 * p + 1):
            idx = uv[j] if k % 2 == 0 else iv[j]
            pltpu.async_copy(
                tables[k].at[idx],
                gbuf.at[p, r, pl.ds((k % 2) * DIM, DIM)], sems[p])
        return carry

      lax.fori_loop(0, _BPW // 16, body, 0)

    for p in range(NUM_TABLES // 2):
      fire(p)

    for p in range(NUM_TABLES // 2):
      pltpu.make_async_copy(
          out_hbm.at[pl.ds(base, _BPW), pl.ds(p * 2 * DIM, 2 * DIM)],
          gbuf.at[p], sems[p]).wait()
      pltpu.async_copy(
          gbuf.at[p],
          out_hbm.at[pl.ds(base, _BPW), pl.ds(p * 2 * DIM, 2 * DIM)], wsem)
    for p in range(NUM_TABLES // 2):
      pltpu.make_async_copy(
          gbuf.at[p],
          out_hbm.at[pl.ds(base, _BPW), pl.ds(p * 2 * DIM, 2 * DIM)],
          wsem).wait()

  return lookup


_lookup = _make()


def kernel(uid, iid, user_table, item_table, src_user_0, src_item_0,
           src_user_1, src_item_1):
  return _lookup(uid.astype(jnp.int32), iid.astype(jnp.int32),
                 user_table, item_table, src_user_0, src_item_0,
                 src_user_1, src_item_1)
